# baseline (speedup 1.0000x reference)
import jax
import jax.numpy as jnp
from jax import lax
from jax.experimental import pallas as pl
from jax.experimental.pallas import tpu as pltpu

N_TYPES = 8
BM = 256


def _fused_body(st_ref, x_ref, wp_ref, bp_ref, wt_ref, bt_ref,
                wc1_ref, bc1_ref, wc2t_ref, bc2_ref, y_ref,
                h1_scr, acc_scr):
    e = pl.program_id(0)
    b = pl.program_id(1)
    rows = pl.ds(b * BM, BM)
    xb = x_ref[rows, :]
    stb = st_ref[rows, :]

    @pl.when(e == 0)
    def _init():
        h1_scr[rows, :] = jnp.maximum(
            jnp.dot(xb, wp_ref[:], preferred_element_type=jnp.float32)
            + bp_ref[:], 0.0)
        acc_scr[rows, :] = xb

    oe = jnp.maximum(
        jnp.dot(h1_scr[rows, :], wt_ref[0],
                preferred_element_type=jnp.float32)
        + bt_ref[pl.ds(e, 1), :], 0.0)
    acc_scr[rows, :] = acc_scr[rows, :] + jnp.where(stb == e, oe, 0.0)

    @pl.when(e == N_TYPES - 1)
    def _contribs():
        h2 = jnp.maximum(
            jnp.dot(acc_scr[rows, :], wc1_ref[:],
                    preferred_element_type=jnp.float32)
            + bc1_ref[:], 0.0)
        y_ref[:] = (jnp.sum(h2 * wc2t_ref[:], axis=1, keepdims=True)
                    + bc2_ref[:])


def kernel(x, sc_types, W_pre, b_pre, W_type, b_type, W_c1, b_c1, W_c2, b_c2):
    d = x.shape[-1]
    xf = x.reshape(-1, d)
    n = xf.shape[0]
    st = sc_types.reshape(-1, 1).astype(jnp.int32)
    nb = n // BM

    y = pl.pallas_call(
        _fused_body,
        grid=(N_TYPES, nb),
        in_specs=[
            pl.BlockSpec((n, 1), lambda e, b: (0, 0)),
            pl.BlockSpec((n, d), lambda e, b: (0, 0)),
            pl.BlockSpec(W_pre.shape, lambda e, b: (0, 0)),
            pl.BlockSpec((1, d), lambda e, b: (0, 0)),
            pl.BlockSpec((1, d, d), lambda e, b: (e, 0, 0)),
            pl.BlockSpec(b_type.shape, lambda e, b: (0, 0)),
            pl.BlockSpec(W_c1.shape, lambda e, b: (0, 0)),
            pl.BlockSpec((1, d), lambda e, b: (0, 0)),
            pl.BlockSpec((1, d), lambda e, b: (0, 0)),
            pl.BlockSpec((1, 1), lambda e, b: (0, 0)),
        ],
        out_specs=pl.BlockSpec((BM, 1), lambda e, b: (b, 0)),
        out_shape=jax.ShapeDtypeStruct((n, 1), jnp.float32),
        scratch_shapes=[
            pltpu.VMEM((n, d), jnp.float32),
            pltpu.VMEM((n, d), jnp.float32),
        ],
    )(st, xf, W_pre, b_pre.reshape(1, -1), W_type, b_type,
      W_c1, b_c1.reshape(1, -1), W_c2.reshape(1, -1), b_c2.reshape(1, 1))
    return y


# speedup vs baseline: 1.6790x; 1.6790x over previous
import jax
import jax.numpy as jnp
from jax.experimental import pallas as pl

N_TYPES = 8
BM = 512


def _fused_body(st_ref, x_ref, wp_ref, bp_ref, wt_ref, bt_ref,
                wc1_ref, bc1_ref, wc2t_ref, bc2_ref, y_ref):
    x = x_ref[:]
    st = st_ref[:]
    h1 = jnp.maximum(
        jnp.dot(x, wp_ref[:], preferred_element_type=jnp.float32)
        + bp_ref[:], 0.0)
    acc = jnp.zeros_like(x)
    for e in range(N_TYPES):
        oe = jnp.maximum(
            jnp.dot(h1, wt_ref[e], preferred_element_type=jnp.float32)
            + bt_ref[e:e + 1, :], 0.0)
        acc = acc + jnp.where(st == e, oe, 0.0)
    xo = x + acc
    h2 = jnp.maximum(
        jnp.dot(xo, wc1_ref[:], preferred_element_type=jnp.float32)
        + bc1_ref[:], 0.0)
    y = jnp.sum(h2 * wc2t_ref[:], axis=1, keepdims=True) + bc2_ref[:]
    y_ref[:] = y


def kernel(x, sc_types, W_pre, b_pre, W_type, b_type, W_c1, b_c1, W_c2, b_c2):
    d = x.shape[-1]
    xf = x.reshape(-1, d)
    n = xf.shape[0]
    st = sc_types.reshape(-1, 1).astype(jnp.int32)
    nb = n // BM

    grid = (nb,)
    y = pl.pallas_call(
        _fused_body,
        grid=grid,
        in_specs=[
            pl.BlockSpec((BM, 1), lambda i: (i, 0)),
            pl.BlockSpec((BM, d), lambda i: (i, 0)),
            pl.BlockSpec(W_pre.shape, lambda i: (0, 0)),
            pl.BlockSpec((1, d), lambda i: (0, 0)),
            pl.BlockSpec(W_type.shape, lambda i: (0, 0, 0)),
            pl.BlockSpec(b_type.shape, lambda i: (0, 0)),
            pl.BlockSpec(W_c1.shape, lambda i: (0, 0)),
            pl.BlockSpec((1, d), lambda i: (0, 0)),
            pl.BlockSpec((1, d), lambda i: (0, 0)),
            pl.BlockSpec((1, 1), lambda i: (0, 0)),
        ],
        out_specs=pl.BlockSpec((BM, 1), lambda i: (i, 0)),
        out_shape=jax.ShapeDtypeStruct((n, 1), jnp.float32),
    )(st, xf, W_pre, b_pre.reshape(1, -1), W_type, b_type,
      W_c1, b_c1.reshape(1, -1), W_c2.reshape(1, -1), b_c2.reshape(1, 1))
    return y


# speedup vs baseline: 1.6805x; 1.0009x over previous
import jax
import jax.numpy as jnp
from jax.experimental import pallas as pl

N_TYPES = 8
BM = 1024


def _fused_body(st_ref, x_ref, wp_ref, bp_ref, wt_ref, bt_ref,
                wc1_ref, bc1_ref, wc2t_ref, bc2_ref, y_ref):
    x = x_ref[:]
    st = st_ref[:]
    h1 = jnp.maximum(
        jnp.dot(x, wp_ref[:], preferred_element_type=jnp.float32)
        + bp_ref[:], 0.0)
    acc = jnp.zeros_like(x)
    for e in range(N_TYPES):
        oe = jnp.maximum(
            jnp.dot(h1, wt_ref[e], preferred_element_type=jnp.float32)
            + bt_ref[e:e + 1, :], 0.0)
        acc = acc + jnp.where(st == e, oe, 0.0)
    xo = x + acc
    h2 = jnp.maximum(
        jnp.dot(xo, wc1_ref[:], preferred_element_type=jnp.float32)
        + bc1_ref[:], 0.0)
    y = jnp.sum(h2 * wc2t_ref[:], axis=1, keepdims=True) + bc2_ref[:]
    y_ref[:] = y


def kernel(x, sc_types, W_pre, b_pre, W_type, b_type, W_c1, b_c1, W_c2, b_c2):
    d = x.shape[-1]
    xf = x.reshape(-1, d)
    n = xf.shape[0]
    st = sc_types.reshape(-1, 1).astype(jnp.int32)
    nb = n // BM

    grid = (nb,)
    y = pl.pallas_call(
        _fused_body,
        grid=grid,
        in_specs=[
            pl.BlockSpec((BM, 1), lambda i: (i, 0)),
            pl.BlockSpec((BM, d), lambda i: (i, 0)),
            pl.BlockSpec(W_pre.shape, lambda i: (0, 0)),
            pl.BlockSpec((1, d), lambda i: (0, 0)),
            pl.BlockSpec(W_type.shape, lambda i: (0, 0, 0)),
            pl.BlockSpec(b_type.shape, lambda i: (0, 0)),
            pl.BlockSpec(W_c1.shape, lambda i: (0, 0)),
            pl.BlockSpec((1, d), lambda i: (0, 0)),
            pl.BlockSpec((1, d), lambda i: (0, 0)),
            pl.BlockSpec((1, 1), lambda i: (0, 0)),
        ],
        out_specs=pl.BlockSpec((BM, 1), lambda i: (i, 0)),
        out_shape=jax.ShapeDtypeStruct((n, 1), jnp.float32),
    )(st, xf, W_pre, b_pre.reshape(1, -1), W_type, b_type,
      W_c1, b_c1.reshape(1, -1), W_c2.reshape(1, -1), b_c2.reshape(1, 1))
    return y


# speedup vs baseline: 1.6848x; 1.0025x over previous
import jax
import jax.numpy as jnp
from jax.experimental import pallas as pl

N_TYPES = 8
BM = 512


def _fused_body(st_ref, x_ref, wp_ref, bp_ref, wt_ref, bt_ref,
                wc1_ref, bc1_ref, wc2t_ref, bc2_ref, y_ref):
    x = x_ref[:]
    st = st_ref[:]
    h1 = jnp.maximum(
        jnp.dot(x, wp_ref[:], preferred_element_type=jnp.float32)
        + bp_ref[:], 0.0)
    acc = jnp.zeros_like(x)
    for e in range(N_TYPES):
        oe = jnp.maximum(
            jnp.dot(h1, wt_ref[e], preferred_element_type=jnp.float32)
            + bt_ref[e:e + 1, :], 0.0)
        acc = acc + jnp.where(st == e, oe, 0.0)
    xo = x + acc
    h2 = jnp.maximum(
        jnp.dot(xo, wc1_ref[:], preferred_element_type=jnp.float32)
        + bc1_ref[:], 0.0)
    y = jnp.sum(h2 * wc2t_ref[:], axis=1, keepdims=True) + bc2_ref[:]
    y_ref[:] = y


def kernel(x, sc_types, W_pre, b_pre, W_type, b_type, W_c1, b_c1, W_c2, b_c2):
    d = x.shape[-1]
    xf = x.reshape(-1, d)
    n = xf.shape[0]
    st = sc_types.reshape(-1, 1).astype(jnp.int32)
    nb = n // BM

    grid = (nb,)
    y = pl.pallas_call(
        _fused_body,
        grid=grid,
        in_specs=[
            pl.BlockSpec((BM, 1), lambda i: (i, 0)),
            pl.BlockSpec((BM, d), lambda i: (i, 0)),
            pl.BlockSpec(W_pre.shape, lambda i: (0, 0)),
            pl.BlockSpec((1, d), lambda i: (0, 0)),
            pl.BlockSpec(W_type.shape, lambda i: (0, 0, 0)),
            pl.BlockSpec(b_type.shape, lambda i: (0, 0)),
            pl.BlockSpec(W_c1.shape, lambda i: (0, 0)),
            pl.BlockSpec((1, d), lambda i: (0, 0)),
            pl.BlockSpec((1, d), lambda i: (0, 0)),
            pl.BlockSpec((1, 1), lambda i: (0, 0)),
        ],
        out_specs=pl.BlockSpec((BM, 1), lambda i: (i, 0)),
        out_shape=jax.ShapeDtypeStruct((n, 1), jnp.float32),
    )(st, xf, W_pre, b_pre.reshape(1, -1), W_type, b_type,
      W_c1, b_c1.reshape(1, -1), W_c2.reshape(1, -1), b_c2.reshape(1, 1))
    return y
